# baseline (speedup 1.0000x reference)
import functools

import jax
import jax.numpy as jnp
from jax import lax
from jax.experimental import pallas as pl
from jax.experimental.pallas import tpu as pltpu
from jax.experimental.pallas import tpu_sc as plsc

J = 0.1
_R = 64
_CW = 4096
_NCH = 12
_CS = _NCH * _CW
_U = 16

_NC, _NS = 2, 16
_NW = _NC * _NS
_L = 16


def _sc_body(x_ref, lab_ref, cor_ref, sum_ref,
             lab_v, win_v, acc_v, gsem, buf0, buf1, s0, s1, sacc_v,
             *, B, V, b_per_w):
    wid = lax.axis_index("s") * _NC + lax.axis_index("c")
    base = wid * b_per_w

    pltpu.sync_copy(lab_ref.at[pl.ds(base, b_per_w)], lab_v)
    lchunks = [lab_v[pl.ds(c * _L, _L)] for c in range(b_per_w // _L)]
    cols = [lchunks[i // _L][i % _L] for i in range(b_per_w)]

    copies = []
    for i in range(b_per_w):
        a = (cols[i] // _L) * _L
        copies.append(
            pltpu.async_copy(x_ref.at[base + i, pl.ds(a, _L)], win_v.at[i], gsem)
        )
    for c in copies:
        c.wait()

    lane = lax.iota(jnp.int32, _L)
    acc = jnp.zeros((_L,), jnp.float32)
    for i in range(b_per_w):
        acc = acc + jnp.where(lane == cols[i] % _L, win_v[i, :], 0.0)
    acc_v[...] = acc
    pltpu.sync_copy(acc_v, cor_ref.at[pl.ds(wid * _L, _L)])

    nrow = B // _NW
    ngrp = nrow // 8
    nch = ngrp * _NCH
    row0 = wid * nrow

    def _start(c, buf, sem):
        g = c // _NCH
        cc = c % _NCH
        return pltpu.async_copy(
            x_ref.at[pl.ds(row0 + g * 8, 8), pl.ds(cc * _CW, _CW)], buf, sem)

    _start(0, buf0, s0)
    _start(1, buf1, s1)

    def _rowsum(buf, r, a):
        def ib(i2, av):
            o = pl.multiple_of(i2 * (_L * _U), _L)
            vs = [buf[r, pl.ds(o + j * _L, _L)] for j in range(_U)]
            while len(vs) > 1:
                nxt = [vs[t] + vs[t + 1] for t in range(0, len(vs) - 1, 2)]
                if len(vs) % 2:
                    nxt.append(vs[-1])
                vs = nxt
            return av + vs[0]

        return lax.fori_loop(0, _CW // (_L * _U), ib, a)

    def _consume(buf, sem, a):
        pltpu.make_async_copy(
            x_ref.at[pl.ds(0, 8), pl.ds(0, _CW)], buf, sem).wait()
        for r in range(8):
            a = _rowsum(buf, r, a)
        return a

    def ob(i, accs):
        a0, a1 = accs
        a0 = _consume(buf0, s0, a0)
        c2 = 2 * i + 2

        @pl.when(c2 < nch)
        def _():
            _start(c2, buf0, s0)

        a1 = _consume(buf1, s1, a1)
        c3 = 2 * i + 3

        @pl.when(c3 < nch)
        def _():
            _start(c3, buf1, s1)

        return (a0, a1)

    z = jnp.zeros((_L,), jnp.float32)
    a0, a1 = lax.fori_loop(0, nch // 2, ob, (z, z))
    sacc_v[...] = a0 + a1
    pltpu.sync_copy(sacc_v, sum_ref.at[pl.ds(wid * _L, _L)])


@functools.cache
def _make_sc(B, V):
    b_per_w = B // _NW
    mesh = plsc.VectorSubcoreMesh(
        core_axis_name="c", subcore_axis_name="s",
        num_cores=_NC, num_subcores=_NS,
    )
    return pl.kernel(
        functools.partial(_sc_body, B=B, V=V, b_per_w=b_per_w),
        out_type=(
            jax.ShapeDtypeStruct((_NW * _L,), jnp.float32),
            jax.ShapeDtypeStruct((_NW * _L,), jnp.float32),
        ),
        mesh=mesh,
        compiler_params=pltpu.CompilerParams(use_tc_tiling_on_sc=True),
        scratch_types=[
            pltpu.VMEM((b_per_w,), jnp.int32),
            pltpu.VMEM((b_per_w, _L), jnp.float32),
            pltpu.VMEM((_L,), jnp.float32),
            pltpu.SemaphoreType.DMA,
            pltpu.VMEM((8, _CW), jnp.float32),
            pltpu.VMEM((8, _CW), jnp.float32),
            pltpu.SemaphoreType.DMA,
            pltpu.SemaphoreType.DMA,
            pltpu.VMEM((_L,), jnp.float32),
        ],
    )


def _tc_body(x_ref, out_ref, acc_ref, *, n_cols):
    r = pl.program_id(0)
    c = pl.program_id(1)
    nr = pl.num_programs(0)
    nc = pl.num_programs(1)

    @pl.when((r == 0) & (c == 0))
    def _init():
        acc_ref[0] = 0.0

    x = x_ref[...]

    @pl.when(c < nc - 1)
    def _full():
        acc_ref[0] += jnp.sum(x)

    @pl.when(c == nc - 1)
    def _masked():
        ids = (c + 1) * _CS + lax.broadcasted_iota(jnp.int32, x.shape, 1)
        acc_ref[0] += jnp.sum(jnp.where(ids < n_cols, x, 0.0))

    @pl.when((r == nr - 1) & (c == nc - 1))
    def _fin():
        out_ref[...] = jnp.reshape(acc_ref[0], (1, 1))


def kernel(outputs, labels):
    B, V = outputs.shape
    lab = labels.astype(jnp.int32)
    cor_parts, sum_parts = _make_sc(B, V)(outputs, lab)

    n_cblocks = (V - _CS + _CS - 1) // _CS
    tc_sum = pl.pallas_call(
        functools.partial(_tc_body, n_cols=V),
        grid=(B // _R, n_cblocks),
        in_specs=[pl.BlockSpec((_R, _CS), lambda r, c: (r, c + 1))],
        out_specs=pl.BlockSpec((1, 1), lambda r, c: (0, 0)),
        out_shape=jax.ShapeDtypeStruct((1, 1), jnp.float32),
        scratch_shapes=[pltpu.SMEM((1,), jnp.float32)],
    )(outputs)

    total = tc_sum[0, 0] + jnp.sum(sum_parts)
    csum = jnp.sum(cor_parts)
    return (total - (V + 1.0) * csum) / B + (2.0 * V - 1.0) * J


# speedup vs baseline: 3.6688x; 3.6688x over previous
import functools

import jax
import jax.numpy as jnp
from jax import lax
from jax.experimental import pallas as pl
from jax.experimental.pallas import tpu as pltpu
from jax.experimental.pallas import tpu_sc as plsc

J = 0.1
_RT = 4000

_NC, _NS = 2, 16
_NW = _NC * _NS
_L = 16


def _sc_body(xt_ref, lab_ref, cor_ref, lab_v, win_v, acc_v, gsem, *, b_per_w):
    wid = lax.axis_index("s") * _NC + lax.axis_index("c")
    base = wid * b_per_w

    pltpu.sync_copy(lab_ref.at[pl.ds(base, b_per_w)], lab_v)
    lchunks = [lab_v[pl.ds(c * _L, _L)] for c in range(b_per_w // _L)]
    rows = [lchunks[i // _L][i % _L] for i in range(b_per_w)]

    copies = []
    for i in range(b_per_w):
        panel = base + (i // _L) * _L
        copies.append(
            pltpu.async_copy(
                xt_ref.at[rows[i], pl.ds(panel, _L)], win_v.at[i], gsem)
        )
    for c in copies:
        c.wait()

    lane = lax.iota(jnp.int32, _L)
    acc = jnp.zeros((_L,), jnp.float32)
    for i in range(b_per_w):
        acc = acc + jnp.where(lane == i % _L, win_v[i, :], 0.0)
    acc_v[...] = acc
    pltpu.sync_copy(acc_v, cor_ref.at[pl.ds(wid * _L, _L)])


@functools.cache
def _make_sc(B, V):
    b_per_w = B // _NW
    mesh = plsc.VectorSubcoreMesh(
        core_axis_name="c", subcore_axis_name="s",
        num_cores=_NC, num_subcores=_NS,
    )
    return pl.kernel(
        functools.partial(_sc_body, b_per_w=b_per_w),
        out_type=jax.ShapeDtypeStruct((_NW * _L,), jnp.float32),
        mesh=mesh,
        compiler_params=pltpu.CompilerParams(use_tc_tiling_on_sc=True),
        scratch_types=[
            pltpu.VMEM((b_per_w,), jnp.int32),
            pltpu.VMEM((b_per_w, _L), jnp.float32),
            pltpu.VMEM((_L,), jnp.float32),
            pltpu.SemaphoreType.DMA,
        ],
    )


def _tc_body(x_ref, out_ref, acc_ref):
    k = pl.program_id(0)

    @pl.when(k == 0)
    def _init():
        acc_ref[0] = 0.0

    acc_ref[0] += jnp.sum(x_ref[...])

    @pl.when(k == pl.num_programs(0) - 1)
    def _fin():
        out_ref[...] = jnp.reshape(acc_ref[0], (1, 1))


def kernel(outputs, labels):
    B, V = outputs.shape
    lab = labels.astype(jnp.int32)
    xt = outputs.T

    cor_parts = _make_sc(B, V)(xt, lab)

    tc_sum = pl.pallas_call(
        _tc_body,
        grid=(V // _RT,),
        in_specs=[pl.BlockSpec((_RT, B), lambda k: (k, 0))],
        out_specs=pl.BlockSpec((1, 1), lambda k: (0, 0)),
        out_shape=jax.ShapeDtypeStruct((1, 1), jnp.float32),
        scratch_shapes=[pltpu.SMEM((1,), jnp.float32)],
    )(xt)

    csum = jnp.sum(cor_parts)
    return (tc_sum[0, 0] - (V + 1.0) * csum) / B + (2.0 * V - 1.0) * J


# speedup vs baseline: 3.7036x; 1.0095x over previous
import functools

import jax
import jax.numpy as jnp
from jax import lax
from jax.experimental import pallas as pl
from jax.experimental.pallas import tpu as pltpu
from jax.experimental.pallas import tpu_sc as plsc

J = 0.1
_RT = 5000

_NC, _NS = 2, 16
_NW = _NC * _NS
_L = 16


def _sc_body(xt_ref, lab_ref, cor_ref, lab_v, win_v, acc_v, gsem, *, b_per_w):
    wid = lax.axis_index("s") * _NC + lax.axis_index("c")
    base = wid * b_per_w

    pltpu.sync_copy(lab_ref.at[pl.ds(base, b_per_w)], lab_v)
    lchunks = [lab_v[pl.ds(c * _L, _L)] for c in range(b_per_w // _L)]
    rows = [lchunks[i // _L][i % _L] for i in range(b_per_w)]

    copies = []
    for i in range(b_per_w):
        panel = base + (i // _L) * _L
        copies.append(
            pltpu.async_copy(
                xt_ref.at[rows[i], pl.ds(panel, _L)], win_v.at[i], gsem)
        )
    for c in copies:
        c.wait()

    lane = lax.iota(jnp.int32, _L)
    acc = jnp.zeros((_L,), jnp.float32)
    for i in range(b_per_w):
        acc = acc + jnp.where(lane == i % _L, win_v[i, :], 0.0)
    acc_v[...] = acc
    pltpu.sync_copy(acc_v, cor_ref.at[pl.ds(wid * _L, _L)])


@functools.cache
def _make_sc(B, V):
    b_per_w = B // _NW
    mesh = plsc.VectorSubcoreMesh(
        core_axis_name="c", subcore_axis_name="s",
        num_cores=_NC, num_subcores=_NS,
    )
    return pl.kernel(
        functools.partial(_sc_body, b_per_w=b_per_w),
        out_type=jax.ShapeDtypeStruct((_NW * _L,), jnp.float32),
        mesh=mesh,
        compiler_params=pltpu.CompilerParams(use_tc_tiling_on_sc=True),
        scratch_types=[
            pltpu.VMEM((b_per_w,), jnp.int32),
            pltpu.VMEM((b_per_w, _L), jnp.float32),
            pltpu.VMEM((_L,), jnp.float32),
            pltpu.SemaphoreType.DMA,
        ],
    )


def _tc_body(x_ref, out_ref, acc_ref):
    k = pl.program_id(0)

    @pl.when(k == 0)
    def _init():
        acc_ref[0] = 0.0

    acc_ref[0] += jnp.sum(x_ref[...])

    @pl.when(k == pl.num_programs(0) - 1)
    def _fin():
        out_ref[...] = jnp.reshape(acc_ref[0], (1, 1))


def kernel(outputs, labels):
    B, V = outputs.shape
    lab = labels.astype(jnp.int32)
    xt = outputs.T

    cor_parts = _make_sc(B, V)(xt, lab)

    tc_sum = pl.pallas_call(
        _tc_body,
        grid=(V // _RT,),
        in_specs=[pl.BlockSpec((_RT, B), lambda k: (k, 0))],
        out_specs=pl.BlockSpec((1, 1), lambda k: (0, 0)),
        out_shape=jax.ShapeDtypeStruct((1, 1), jnp.float32),
        scratch_shapes=[pltpu.SMEM((1,), jnp.float32)],
    )(xt)

    csum = jnp.sum(cor_parts)
    return (tc_sum[0, 0] - (V + 1.0) * csum) / B + (2.0 * V - 1.0) * J
